# baseline (speedup 1.0000x reference)
import functools

import jax
import jax.numpy as jnp
from jax import lax
from jax.experimental import pallas as pl
from jax.experimental.pallas import tpu as pltpu
from jax.experimental.pallas import tpu_sc as plsc

NC = 2
NS = 16
NW = NC * NS
LANES = 16
CHUNK = 4096
NBUF = 2


@functools.lru_cache(maxsize=None)
def _lookup_fn(R, V1, B, D):
    RPW = R // NW
    mesh = plsc.VectorSubcoreMesh(core_axis_name="c", subcore_axis_name="s")

    @functools.partial(
        pl.kernel,
        out_type=jax.ShapeDtypeStruct((R, B), jnp.float32),
        mesh=mesh,
        scratch_types=[
            pltpu.VMEM((V1,), jnp.float32),
            pltpu.VMEM((B,), jnp.int32),
            pltpu.VMEM((NBUF, CHUNK), jnp.float32),
            pltpu.SemaphoreType.DMA,
            pltpu.SemaphoreType.DMA,
        ],
        compiler_params=pltpu.CompilerParams(needs_layout_passes=False),
    )
    def body(tabT_ref, xT_ref, out_ref, row_v, xcol_v, oc_v, sem, sem_o):
        w = lax.axis_index("s") * NC + lax.axis_index("c")
        r0 = w * RPW
        rend = r0 + RPW
        f0 = r0 // D
        split = jnp.minimum((f0 + 1) * D, rend)

        def drain_one(r):
            pltpu.make_async_copy(
                oc_v.at[0], out_ref.at[r, pl.ds(0, CHUNK)], sem_o
            ).wait()

        def do_rows(f, lo, hi):
            pltpu.sync_copy(xT_ref.at[f], xcol_v)

            @pl.loop(lo, hi)
            def _row(r):
                pltpu.async_copy(tabT_ref.at[r], row_v, sem).wait()
                for c in range(B // CHUNK):
                    buf = c % NBUF

                    if c < NBUF:
                        @pl.when(r > r0)
                        def _():
                            drain_one(r)
                    else:
                        drain_one(r)

                    @plsc.parallel_loop(0, CHUNK // LANES, unroll=32)
                    def _grp(k):
                        b0 = k * LANES
                        idx = xcol_v[pl.ds(c * CHUNK + b0, LANES)]
                        oc_v[buf, pl.ds(b0, LANES)] = plsc.load_gather(
                            row_v, [idx]
                        )

                    pltpu.async_copy(
                        oc_v.at[buf],
                        out_ref.at[r, pl.ds(c * CHUNK, CHUNK)],
                        sem_o,
                    )

        do_rows(f0, r0, split)

        @pl.when(split < rend)
        def _():
            do_rows(f0 + 1, split, rend)

        for _ in range(NBUF):
            drain_one(r0)

    return body


def _mask_body(x_ref, o_ref):
    o_ref[...] = x_ref[...] == 0


def kernel(x, tables):
    F, V1, D = tables.shape
    B = x.shape[0]
    tabT = tables.transpose(0, 2, 1).reshape(F * D, V1)
    xT = x.T
    outT = _lookup_fn(F * D, V1, B, D)(tabT, xT)
    emb = outT.reshape(F, D, B).transpose(2, 0, 1)
    maskT = pl.pallas_call(
        _mask_body,
        out_shape=jax.ShapeDtypeStruct((F, B), jnp.bool_),
    )(xT)
    return emb, maskT.T


# speedup vs baseline: 1.0472x; 1.0472x over previous
import functools

import jax
import jax.numpy as jnp
from jax import lax
from jax.experimental import pallas as pl
from jax.experimental.pallas import tpu as pltpu
from jax.experimental.pallas import tpu_sc as plsc

NC = 2
NS = 16
NW = NC * NS
LANES = 16
CHUNK = 2048
NBUF = 4


@functools.lru_cache(maxsize=None)
def _lookup_fn(R, V1, B, D):
    RPW = R // NW
    mesh = plsc.VectorSubcoreMesh(core_axis_name="c", subcore_axis_name="s")

    @functools.partial(
        pl.kernel,
        out_type=jax.ShapeDtypeStruct((R, B), jnp.float32),
        mesh=mesh,
        scratch_types=[
            pltpu.VMEM((V1,), jnp.float32),
            pltpu.VMEM((B,), jnp.int32),
            pltpu.VMEM((NBUF, CHUNK), jnp.float32),
            pltpu.SemaphoreType.DMA,
            pltpu.SemaphoreType.DMA,
        ],
        compiler_params=pltpu.CompilerParams(needs_layout_passes=False),
    )
    def body(tabT_ref, xT_ref, out_ref, row_v, xcol_v, oc_v, sem, sem_o):
        w = lax.axis_index("s") * NC + lax.axis_index("c")
        r0 = w * RPW
        rend = r0 + RPW
        f0 = r0 // D
        split = jnp.minimum((f0 + 1) * D, rend)

        def drain_one(r):
            pltpu.make_async_copy(
                oc_v.at[0], out_ref.at[r, pl.ds(0, CHUNK)], sem_o
            ).wait()

        def do_rows(f, lo, hi):
            pltpu.sync_copy(xT_ref.at[f], xcol_v)

            @pl.loop(lo, hi)
            def _row(r):
                cp = pltpu.async_copy(tabT_ref.at[r], row_v, sem)

                @pl.when(r > r0)
                def _():
                    for _ in range(NBUF):
                        drain_one(r)

                cp.wait()
                for c in range(B // CHUNK):
                    buf = c % NBUF
                    if c >= NBUF:
                        drain_one(r)

                    @plsc.parallel_loop(0, CHUNK // LANES, unroll=16)
                    def _grp(k):
                        b0 = k * LANES
                        idx = xcol_v[pl.ds(c * CHUNK + b0, LANES)]
                        oc_v[buf, pl.ds(b0, LANES)] = plsc.load_gather(
                            row_v, [idx]
                        )

                    pltpu.async_copy(
                        oc_v.at[buf],
                        out_ref.at[r, pl.ds(c * CHUNK, CHUNK)],
                        sem_o,
                    )

        do_rows(f0, r0, split)

        @pl.when(split < rend)
        def _():
            do_rows(f0 + 1, split, rend)

        for _ in range(NBUF):
            drain_one(r0)

    return body


def _mask_body(x_ref, o_ref):
    o_ref[...] = x_ref[...] == 0


def kernel(x, tables):
    F, V1, D = tables.shape
    B = x.shape[0]
    tabT = tables.transpose(0, 2, 1).reshape(F * D, V1)
    xT = x.T
    outT = _lookup_fn(F * D, V1, B, D)(tabT, xT)
    emb = outT.reshape(F, D, B).transpose(2, 0, 1)
    maskT = pl.pallas_call(
        _mask_body,
        out_shape=jax.ShapeDtypeStruct((F, B), jnp.bool_),
    )(xT)
    return emb, maskT.T
